# baseline (speedup 1.0000x reference)
import functools

import jax
import jax.numpy as jnp
from jax import lax
from jax.experimental import pallas as pl
from jax.experimental.pallas import tpu as pltpu
from jax.experimental.pallas import tpu_sc as plsc

VOCAB = 100000
EMB = 64
NPROP = 26
BATCH = 16384

NC = 2
NS = 16
NW = NC * NS
B_W = BATCH // NW
ROWS_C = 8
NCHUNK = B_W // ROWS_C
NBUF = 4
LANES = 16

GROUPS = (7, 7, 6, 6)


def _make_group_call(np_g):
    idxn = np_g * B_W
    chunk = ROWS_C * np_g

    def fire(table_hbm, idx_v, buf, sem, k):
        pltpu.async_copy(
            table_hbm.at[idx_v.at[pl.ds(k * chunk, chunk)]],
            buf,
            sem,
        )

    def drain(table_hbm, buf, sem):
        pltpu.make_async_copy(table_hbm.at[pl.ds(0, chunk)], buf, sem).wait()

    def accumulate(acc_v, buf, k):
        for b in range(ROWS_C):
            for c in range(EMB // LANES):
                sl = pl.ds(c * LANES, LANES)
                s = buf[b * np_g, sl]
                for r in range(1, np_g):
                    s = s + buf[b * np_g + r, sl]
                acc_v[ROWS_C * k + b, sl] = s

    def body(propf_hbm, table3_hbm, out_hbm, idx_v,
             b0, b1, b2, b3, acc_v, s0, s1, s2, s3):
        bufs = [b0, b1, b2, b3]
        sems = [s0, s1, s2, s3]
        wid = lax.axis_index("s") * NC + lax.axis_index("c")
        table_hbm = table3_hbm.at[0]

        pltpu.sync_copy(propf_hbm.at[pl.ds(wid * idxn, idxn)], idx_v)

        iota = lax.iota(jnp.int32, LANES)

        @plsc.parallel_loop(0, idxn // LANES, unroll=4)
        def _(kk):
            pos = kk * LANES + iota
            off = lax.rem(pos, jnp.int32(np_g)) * jnp.int32(VOCAB)
            sl = pl.ds(kk * LANES, LANES)
            idx_v[sl] = idx_v[sl] + off

        for s in range(NBUF):
            fire(table_hbm, idx_v, bufs[s], sems[s], jnp.int32(s))

        def kk_step(kk, _):
            for s in range(NBUF):
                k = NBUF * kk + s
                drain(table_hbm, bufs[s], sems[s])
                accumulate(acc_v, bufs[s], k)

                @pl.when(kk < NCHUNK // NBUF - 1)
                def _():
                    fire(table_hbm, idx_v, bufs[s], sems[s], k + NBUF)

            return 0

        lax.fori_loop(0, NCHUNK // NBUF, kk_step, 0)
        pltpu.sync_copy(acc_v, out_hbm.at[pl.ds(wid * B_W, B_W)])

    mesh = plsc.VectorSubcoreMesh(core_axis_name="c", subcore_axis_name="s")
    return functools.partial(
        pl.kernel,
        out_type=jax.ShapeDtypeStruct((BATCH, EMB), jnp.float32),
        mesh=mesh,
        scratch_types=[
            pltpu.VMEM((idxn,), jnp.int32),
            pltpu.VMEM((chunk, EMB), jnp.float32),
            pltpu.VMEM((chunk, EMB), jnp.float32),
            pltpu.VMEM((chunk, EMB), jnp.float32),
            pltpu.VMEM((chunk, EMB), jnp.float32),
            pltpu.VMEM((B_W, EMB), jnp.float32),
            pltpu.SemaphoreType.DMA,
            pltpu.SemaphoreType.DMA,
            pltpu.SemaphoreType.DMA,
            pltpu.SemaphoreType.DMA,
        ],
        compiler_params=pltpu.CompilerParams(use_tc_tiling_on_sc=False),
    )(body)


def kernel(prop, tables):
    prop32 = prop.astype(jnp.int32)
    partials = []
    p0 = 0
    for np_g in GROUPS:
        propf = prop32[:, p0:p0 + np_g].reshape(-1)
        tab_g = tables[p0:p0 + np_g]
        partials.append(_make_group_call(np_g)(propf, tab_g))
        p0 += np_g
    out = partials[0] + partials[1] + partials[2] + partials[3]
    return out[:, None, :]
